# baseline (speedup 1.0000x reference)
import functools

import jax
import jax.numpy as jnp
import numpy as np
from jax import lax
from jax.experimental import pallas as pl
from jax.experimental.pallas import tpu as pltpu
from jax.experimental.pallas import tpu_sc as plsc

TOPK = 64
_I32MIN = np.int32(-2147483648)
_CHUNK = 16
_UNIT = 128
_NWORK = 32


def _key_of(a):
    u = lax.bitcast_convert_type(a, jnp.int32)
    return jnp.where(u >= 0, u, u ^ jnp.int32(0x7FFFFFFF))


def _unkey(ks):
    u = jnp.where(ks >= 0, ks, ks ^ jnp.int32(0x7FFFFFFF))
    return lax.bitcast_convert_type(u, jnp.float32)


def _enc_body(x_ref, w_ref, b_ref, db_ref, out_ref, cmax_ref):
    xb = (x_ref[...] - db_ref[...]).astype(jnp.bfloat16)
    aw = lax.dot_general(
        xb, w_ref[...], (((1,), (1,)), ((), ())),
        preferred_element_type=jnp.float32,
    )
    a = aw + b_ref[...]
    bm, bn = a.shape
    out_ref[...] = a.reshape(bm, bn // _UNIT, _UNIT)
    cmax_ref[...] = jnp.max(a.reshape(bm, bn // _CHUNK, _CHUNK), axis=2)


def _lsel_body(cm_ref, lv_ref, *, k):
    cm = cm_ref[...]
    ks = _key_of(cm)
    kk = ks ^ _I32MIN
    bm = ks.shape[0]
    prefix0 = jnp.zeros((bm, 1), jnp.int32)
    krem0 = jnp.full((bm, 1), k, jnp.int32)

    def body(i, carry):
        prefix, krem = carry
        b = 31 - i
        elem_hi = lax.shift_right_logical(kk, b)
        cand = lax.shift_right_logical(prefix, b) | jnp.int32(1)
        m = elem_hi == cand
        cnt = jnp.sum(m.astype(jnp.int32), axis=1, keepdims=True)
        take = cnt >= krem
        bit = lax.shift_left(jnp.int32(1), b)
        prefix = jnp.where(take, prefix | bit, prefix)
        krem = jnp.where(take, krem, krem - cnt)
        return prefix, krem

    prefix, _ = lax.fori_loop(0, 32, body, (prefix0, krem0))
    lv_ref[...] = prefix ^ _I32MIN


def _sc_topk_body(a5_hbm, cm_hbm, lv_hbm, out_hbm, cmbuf, cidbuf,
                  gath2d, candbuf, lvbuf, histbuf, outvbuf, gsem,
                  *, nunit, rpw, k):
    coreid = lax.axis_index("c")
    sid = lax.axis_index("s")
    wid = sid * 2 + coreid
    base = wid * rpw
    pltpu.sync_copy(lv_hbm.at[pl.ds(base, rpw)], lvbuf.at[pl.ds(0, rpw)])
    iota16 = lax.iota(jnp.int32, 16)
    ones16 = jnp.ones((16,), jnp.int32)
    big = jnp.int32(2147483647)
    for j in range(nunit // 16):
        cidbuf[pl.ds(16 * j, 16)] = jnp.zeros((16,), jnp.int32)

    iota8 = iota16 * 8

    def do_row(r, _):
        pltpu.sync_copy(cm_hbm.at[base + r], cmbuf)
        lval = lvbuf[pl.ds(r, 16)][0]
        lv = jnp.full((16,), lval, jnp.float32)
        gbase = (base + r) * nunit

        def scan_a(i, off_v):
            v = plsc.load_gather(cmbuf, [iota8 + i * 128])
            for g in range(1, 8):
                v = jnp.maximum(
                    v, plsc.load_gather(cmbuf, [iota8 + (i * 128 + g)]))
            m = v > lv
            gcid = iota16 + (gbase + i * 16)
            c = plsc.cumsum(m.astype(jnp.int32))
            plsc.store_scatter(cidbuf, [off_v + c - 1], gcid, mask=m)
            return off_v + plsc.all_reduce_population_count(m)

        offc = lax.fori_loop(0, nunit // 16, scan_a,
                             jnp.zeros((16,), jnp.int32))
        ncu = jnp.max(offc)
        nt = (ncu + 15) // 16

        def fire(t, _):
            pltpu.async_copy(a5_hbm.at[cidbuf.at[pl.ds(t * 16, 16)]],
                             gath2d.at[pl.ds(t * 16, 16)], gsem)
            return 0

        lax.fori_loop(0, nt, fire, 0)

        def drain(t, _):
            pltpu.make_async_copy(
                a5_hbm.at[cidbuf.at[pl.ds(t * 16, 16)]],
                gath2d.at[pl.ds(t * 16, 16)], gsem).wait()
            return 0

        lax.fori_loop(0, nt, drain, 0)

        filler = plsc.bitcast(jnp.full((16,), -1, jnp.int32), jnp.float32)

        def scan_b(q, off_v):
            for j8 in range(8):
                v = gath2d[q, pl.ds(j8 * 16, 16)]
                m = v > lv
                candbuf[pl.ds(q * 128 + j8 * 16, 16)] = jnp.where(m, v, filler)
                off_v = off_v + plsc.all_reduce_population_count(m)
            return off_v

        off_v = lax.fori_loop(0, ncu, scan_b, jnp.zeros((16,), jnp.int32))
        n0 = jnp.max(off_v)

        n = ncu * 128
        nvq = ncu * 8
        kth = jnp.int32(k)
        tkk = jnp.int32(0)
        for s in range(4):
            sh = 24 - 8 * s
            for g in range(16):
                histbuf[pl.ds(16 * g, 16)] = jnp.zeros((16,), jnp.int32)

            def hist_i(i, _, sh=sh):
                v = candbuf[pl.ds(i * 16, 16)]
                kk = _key_of(v) ^ _I32MIN
                b = lax.shift_right_logical(kk, sh) & jnp.int32(0xFF)
                plsc.addupdate_scatter(histbuf, [b], ones16)
                return 0

            lax.fori_loop(0, nvq, hist_i, 0)

            nk = n - kth
            bsum = jnp.int32(0)
            sb = big
            ctot = jnp.int32(0)
            for g in range(16):
                h = histbuf[pl.ds(16 * g, 16)]
                cs = plsc.cumsum(h) + ctot
                le = cs <= nk
                bsum = bsum + jnp.max(plsc.all_reduce_population_count(le))
                sb = jnp.minimum(sb, jnp.min(jnp.where(le, big, cs)))
                ctot = jnp.max(cs)
            bbin = bsum
            kth = kth - (n - sb)
            tkk = tkk | lax.shift_left(bbin, sh)
            if s < 3:
                bb = jnp.full((16,), bbin, jnp.int32)

                def rc_i(i, _, sh=sh, bb=bb):
                    v = candbuf[pl.ds(i * 16, 16)]
                    kk = _key_of(v) ^ _I32MIN
                    b = lax.shift_right_logical(kk, sh) & jnp.int32(0xFF)
                    candbuf[pl.ds(i * 16, 16)] = jnp.where(b == bb, v, filler)
                    return 0

                lax.fori_loop(0, nvq, rc_i, 0)

        t_sel = tkk ^ _I32MIN
        t_fb = jnp.max(_key_of(lv))
        tout = jnp.where(n0 >= k, t_sel, t_fb)
        plsc.store_scatter(outvbuf, [jnp.full((16,), r, jnp.int32)],
                           jnp.full((16,), tout, jnp.int32),
                           mask=iota16 == 0)
        return 0

    lax.fori_loop(0, rpw, do_row, 0)
    pltpu.sync_copy(outvbuf, out_hbm.at[pl.ds(base, rpw)])


def _dec_body(a_ref, t_ref, w_ref, db_ref, f_ref, xhat_ref, acc_ref, *, nk):
    j = pl.program_id(1)
    a3 = a_ref[...]
    a = a3.reshape(a3.shape[0], a3.shape[1] * a3.shape[2])
    ks = _key_of(a)
    m = ks >= t_ref[...]
    f = jnp.maximum(jnp.where(m, a, 0.0), 0.0)
    f_ref[...] = f
    partial = lax.dot_general(
        f.astype(jnp.bfloat16), w_ref[...], (((1,), (1,)), ((), ())),
        preferred_element_type=jnp.float32,
    )

    @pl.when(j == 0)
    def _():
        acc_ref[...] = partial

    @pl.when(j > 0)
    def _():
        acc_ref[...] += partial

    @pl.when(j == nk - 1)
    def _():
        xhat_ref[...] = acc_ref[...] + db_ref[...]


def kernel(x, enc_w, enc_b, dec_w, dec_b):
    ntok, vec = x.shape
    lat = enc_w.shape[0]
    nchunk = lat // _CHUNK

    bm = 256 if ntok % 256 == 0 else ntok
    bn = 2048 if lat % 2048 == 0 else lat
    n_lat = lat // bn

    enc_b2 = enc_b.reshape(1, lat)
    dec_b2 = dec_b.reshape(1, vec)
    enc_wh = enc_w.astype(jnp.bfloat16)
    dec_wh = dec_w.astype(jnp.bfloat16)

    nunit = lat // _UNIT
    a, cmax = pl.pallas_call(
        _enc_body,
        grid=(ntok // bm, n_lat),
        in_specs=[
            pl.BlockSpec((bm, vec), lambda i, j: (i, 0)),
            pl.BlockSpec((bn, vec), lambda i, j: (j, 0)),
            pl.BlockSpec((1, bn), lambda i, j: (0, j)),
            pl.BlockSpec((1, vec), lambda i, j: (0, 0)),
        ],
        out_specs=[
            pl.BlockSpec((bm, bn // _UNIT, _UNIT), lambda i, j: (i, j, 0)),
            pl.BlockSpec((bm, bn // _CHUNK), lambda i, j: (i, j)),
        ],
        out_shape=[
            jax.ShapeDtypeStruct((ntok, nunit, _UNIT), jnp.float32),
            jax.ShapeDtypeStruct((ntok, nchunk), jnp.float32),
        ],
    )(x, enc_wh, enc_b2, dec_b2)

    lkeys = pl.pallas_call(
        functools.partial(_lsel_body, k=TOPK + 1),
        grid=(1,),
        in_specs=[pl.BlockSpec((ntok, nchunk), lambda i: (0, 0))],
        out_specs=pl.BlockSpec((ntok, 1), lambda i: (0, 0)),
        out_shape=jax.ShapeDtypeStruct((ntok, 1), jnp.int32),
    )(cmax)
    lvals = _unkey(lkeys.reshape(ntok))

    rpw = ntok // _NWORK
    a5 = a.reshape(ntok * nunit, _UNIT)
    mesh = plsc.VectorSubcoreMesh(core_axis_name="c", subcore_axis_name="s")
    tkeys = pl.kernel(
        functools.partial(_sc_topk_body, nunit=nunit, rpw=rpw, k=TOPK),
        out_type=jax.ShapeDtypeStruct((ntok,), jnp.int32),
        mesh=mesh,
        compiler_params=pltpu.CompilerParams(needs_layout_passes=False),
        scratch_types=[
            pltpu.VMEM((nchunk,), jnp.float32),
            pltpu.VMEM((nunit,), jnp.int32),
            pltpu.VMEM((nunit, _UNIT), jnp.float32),
            pltpu.VMEM((lat,), jnp.float32),
            pltpu.VMEM((rpw + 16,), jnp.float32),
            pltpu.VMEM((256,), jnp.int32),
            pltpu.VMEM((rpw,), jnp.int32),
            pltpu.SemaphoreType.DMA,
        ],
    )(a5, cmax, lvals)
    tkeys2 = tkeys.reshape(ntok, 1)

    bk = 2048 if lat % 2048 == 0 else lat
    nk = lat // bk
    f, xhat = pl.pallas_call(
        functools.partial(_dec_body, nk=nk),
        grid=(ntok // bm, nk),
        in_specs=[
            pl.BlockSpec((bm, bk // _UNIT, _UNIT), lambda i, j: (i, j, 0)),
            pl.BlockSpec((bm, 1), lambda i, j: (i, 0)),
            pl.BlockSpec((vec, bk), lambda i, j: (0, j)),
            pl.BlockSpec((1, vec), lambda i, j: (0, 0)),
        ],
        out_specs=[
            pl.BlockSpec((bm, bk), lambda i, j: (i, j)),
            pl.BlockSpec((bm, vec), lambda i, j: (i, 0)),
        ],
        out_shape=[
            jax.ShapeDtypeStruct((ntok, lat), jnp.float32),
            jax.ShapeDtypeStruct((ntok, vec), jnp.float32),
        ],
        scratch_shapes=[pltpu.VMEM((bm, vec), jnp.float32)],
    )(a, tkeys2, dec_wh, dec_b2)

    return (f, xhat)


# speedup vs baseline: 2.1939x; 2.1939x over previous
import functools

import jax
import jax.numpy as jnp
import numpy as np
from jax import lax
from jax.experimental import pallas as pl
from jax.experimental.pallas import tpu as pltpu
from jax.experimental.pallas import tpu_sc as plsc

TOPK = 64
_I32MIN = np.int32(-2147483648)
_CHUNK = 16
_UNIT = 128
_NWORK = 32


def _key_of(a):
    u = lax.bitcast_convert_type(a, jnp.int32)
    return jnp.where(u >= 0, u, u ^ jnp.int32(0x7FFFFFFF))


def _unkey(ks):
    u = jnp.where(ks >= 0, ks, ks ^ jnp.int32(0x7FFFFFFF))
    return lax.bitcast_convert_type(u, jnp.float32)


def _enc_body(x_ref, w_ref, b_ref, db_ref, out_ref, cmax_ref):
    xb = (x_ref[...] - db_ref[...]).astype(jnp.bfloat16)
    aw = lax.dot_general(
        xb, w_ref[...], (((1,), (1,)), ((), ())),
        preferred_element_type=jnp.float32,
    )
    a = aw + b_ref[...]
    bm, bn = a.shape
    out_ref[...] = a.reshape(bm, bn // _UNIT, _UNIT)
    cmax_ref[...] = jnp.max(a.reshape(bm, bn // _CHUNK, _CHUNK), axis=2)


def _lsel_body(cm_ref, lv_ref, *, k):
    cm = cm_ref[...]
    ks = _key_of(cm)
    kk = ks ^ _I32MIN
    bm = ks.shape[0]
    prefix0 = jnp.zeros((bm, 1), jnp.int32)
    krem0 = jnp.full((bm, 1), k, jnp.int32)

    def body(i, carry):
        prefix, krem = carry
        b = 31 - i
        elem_hi = lax.shift_right_logical(kk, b)
        cand = lax.shift_right_logical(prefix, b) | jnp.int32(1)
        m = elem_hi == cand
        cnt = jnp.sum(m.astype(jnp.int32), axis=1, keepdims=True)
        take = cnt >= krem
        bit = lax.shift_left(jnp.int32(1), b)
        prefix = jnp.where(take, prefix | bit, prefix)
        krem = jnp.where(take, krem, krem - cnt)
        return prefix, krem

    prefix, _ = lax.fori_loop(0, 32, body, (prefix0, krem0))
    lv_ref[...] = prefix ^ _I32MIN


def _sc_topk_body(a5_hbm, cm_hbm, lv_hbm, out_hbm, cmbuf, cidbuf,
                  gath2d, candbuf, candbuf2, lvbuf, histbuf, outvbuf, gsem,
                  *, nunit, rpw, k):
    coreid = lax.axis_index("c")
    sid = lax.axis_index("s")
    wid = sid * 2 + coreid
    base = wid * rpw
    pltpu.sync_copy(lv_hbm.at[pl.ds(base, rpw)], lvbuf.at[pl.ds(0, rpw)])
    iota16 = lax.iota(jnp.int32, 16)
    ones16 = jnp.ones((16,), jnp.int32)
    big = jnp.int32(2147483647)
    for j in range(nunit // 16):
        cidbuf[pl.ds(16 * j, 16)] = jnp.zeros((16,), jnp.int32)

    iota8 = iota16 * 8

    def do_row(r, _):
        pltpu.sync_copy(cm_hbm.at[base + r], cmbuf)
        lval = lvbuf[pl.ds(r, 16)][0]
        lv = jnp.full((16,), lval, jnp.float32)
        gbase = (base + r) * nunit

        def scan_a(i, off_v):
            v = plsc.load_gather(cmbuf, [iota8 + i * 128])
            for g in range(1, 8):
                v = jnp.maximum(
                    v, plsc.load_gather(cmbuf, [iota8 + (i * 128 + g)]))
            m = v > lv
            gcid = iota16 + (gbase + i * 16)
            c = plsc.cumsum(m.astype(jnp.int32))
            plsc.store_scatter(cidbuf, [off_v + c - 1], gcid, mask=m)
            return off_v + plsc.all_reduce_population_count(m)

        offc = lax.fori_loop(0, nunit // 16, scan_a,
                             jnp.zeros((16,), jnp.int32))
        ncu = jnp.max(offc)
        nt = (ncu + 15) // 16

        def fire(t, _):
            pltpu.async_copy(a5_hbm.at[cidbuf.at[pl.ds(t * 16, 16)]],
                             gath2d.at[pl.ds(t * 16, 16)], gsem)
            return 0

        lax.fori_loop(0, nt, fire, 0)

        def drain(t, _):
            pltpu.make_async_copy(
                a5_hbm.at[cidbuf.at[pl.ds(t * 16, 16)]],
                gath2d.at[pl.ds(t * 16, 16)], gsem).wait()
            return 0

        lax.fori_loop(0, nt, drain, 0)

        def scan_b(q, off_v):
            for j8 in range(8):
                v = gath2d[q, pl.ds(j8 * 16, 16)]
                m = v > lv
                c = plsc.cumsum(m.astype(jnp.int32))
                plsc.store_scatter(candbuf, [off_v + c - 1], v, mask=m)
                off_v = off_v + plsc.all_reduce_population_count(m)
            return off_v

        off_v = lax.fori_loop(0, ncu, scan_b, jnp.zeros((16,), jnp.int32))
        n0 = jnp.max(off_v)

        bufs = [candbuf, candbuf2, candbuf, candbuf2]
        n = n0
        kth = jnp.int32(k)
        tkk = jnp.int32(0)
        for s in range(4):
            sh = 24 - 8 * s
            src = bufs[s]
            for g in range(16):
                histbuf[pl.ds(16 * g, 16)] = jnp.zeros((16,), jnp.int32)
            nv = (n + 15) // 16

            def hist_i(i, _, src=src, sh=sh, n=n):
                v = src[pl.ds(i * 16, 16)]
                kk = _key_of(v) ^ _I32MIN
                b = lax.shift_right_logical(kk, sh) & jnp.int32(0xFF)
                valid = iota16 < (n - i * 16)
                plsc.addupdate_scatter(histbuf, [b], ones16, mask=valid)
                return 0

            lax.fori_loop(0, nv, hist_i, 0)

            nk = n - kth
            bsum = jnp.int32(0)
            sb = big
            ctot = jnp.int32(0)
            for g in range(16):
                h = histbuf[pl.ds(16 * g, 16)]
                cs = plsc.cumsum(h) + ctot
                le = cs <= nk
                bsum = bsum + jnp.max(plsc.all_reduce_population_count(le))
                sb = jnp.minimum(sb, jnp.min(jnp.where(le, big, cs)))
                ctot = jnp.max(cs)
            bbin = bsum
            kth = kth - (n - sb)
            tkk = tkk | lax.shift_left(bbin, sh)
            if s < 3:
                dst = bufs[s + 1]
                bb = jnp.full((16,), bbin, jnp.int32)

                def rc_i(i, off, src=src, sh=sh, n=n, bb=bb, dst=dst):
                    v = src[pl.ds(i * 16, 16)]
                    kk = _key_of(v) ^ _I32MIN
                    b = lax.shift_right_logical(kk, sh) & jnp.int32(0xFF)
                    valid = iota16 < (n - i * 16)
                    m = valid & (b == bb)
                    c = plsc.cumsum(m.astype(jnp.int32))
                    plsc.store_scatter(dst, [off + c - 1], v, mask=m)
                    return off + plsc.all_reduce_population_count(m)

                offd = lax.fori_loop(0, nv, rc_i,
                                     jnp.zeros((16,), jnp.int32))
                n = jnp.max(offd)

        t_sel = tkk ^ _I32MIN
        t_fb = jnp.max(_key_of(lv))
        tout = jnp.where(n0 >= k, t_sel, t_fb)
        plsc.store_scatter(outvbuf, [jnp.full((16,), r, jnp.int32)],
                           jnp.full((16,), tout, jnp.int32),
                           mask=iota16 == 0)
        return 0

    lax.fori_loop(0, rpw, do_row, 0)
    pltpu.sync_copy(outvbuf, out_hbm.at[pl.ds(base, rpw)])


def _dec_body(a_ref, t_ref, w_ref, db_ref, f_ref, xhat_ref, acc_ref, *, nk):
    j = pl.program_id(1)
    a3 = a_ref[...]
    a = a3.reshape(a3.shape[0], a3.shape[1] * a3.shape[2])
    ks = _key_of(a)
    m = ks >= t_ref[...]
    f = jnp.maximum(jnp.where(m, a, 0.0), 0.0)
    f_ref[...] = f
    partial = lax.dot_general(
        f.astype(jnp.bfloat16), w_ref[...], (((1,), (1,)), ((), ())),
        preferred_element_type=jnp.float32,
    )

    @pl.when(j == 0)
    def _():
        acc_ref[...] = partial

    @pl.when(j > 0)
    def _():
        acc_ref[...] += partial

    @pl.when(j == nk - 1)
    def _():
        xhat_ref[...] = acc_ref[...] + db_ref[...]


def kernel(x, enc_w, enc_b, dec_w, dec_b):
    ntok, vec = x.shape
    lat = enc_w.shape[0]
    nchunk = lat // _CHUNK

    bm = 256 if ntok % 256 == 0 else ntok
    bn = 2048 if lat % 2048 == 0 else lat
    n_lat = lat // bn

    enc_b2 = enc_b.reshape(1, lat)
    dec_b2 = dec_b.reshape(1, vec)
    enc_wh = enc_w.astype(jnp.bfloat16)
    dec_wh = dec_w.astype(jnp.bfloat16)

    nunit = lat // _UNIT
    a, cmax = pl.pallas_call(
        _enc_body,
        grid=(ntok // bm, n_lat),
        in_specs=[
            pl.BlockSpec((bm, vec), lambda i, j: (i, 0)),
            pl.BlockSpec((bn, vec), lambda i, j: (j, 0)),
            pl.BlockSpec((1, bn), lambda i, j: (0, j)),
            pl.BlockSpec((1, vec), lambda i, j: (0, 0)),
        ],
        out_specs=[
            pl.BlockSpec((bm, bn // _UNIT, _UNIT), lambda i, j: (i, j, 0)),
            pl.BlockSpec((bm, bn // _CHUNK), lambda i, j: (i, j)),
        ],
        out_shape=[
            jax.ShapeDtypeStruct((ntok, nunit, _UNIT), jnp.float32),
            jax.ShapeDtypeStruct((ntok, nchunk), jnp.float32),
        ],
    )(x, enc_wh, enc_b2, dec_b2)

    lkeys = pl.pallas_call(
        functools.partial(_lsel_body, k=TOPK + 1),
        grid=(1,),
        in_specs=[pl.BlockSpec((ntok, nchunk), lambda i: (0, 0))],
        out_specs=pl.BlockSpec((ntok, 1), lambda i: (0, 0)),
        out_shape=jax.ShapeDtypeStruct((ntok, 1), jnp.int32),
    )(cmax)
    lvals = _unkey(lkeys.reshape(ntok))

    rpw = ntok // _NWORK
    a5 = a.reshape(ntok * nunit, _UNIT)
    mesh = plsc.VectorSubcoreMesh(core_axis_name="c", subcore_axis_name="s")
    tkeys = pl.kernel(
        functools.partial(_sc_topk_body, nunit=nunit, rpw=rpw, k=TOPK),
        out_type=jax.ShapeDtypeStruct((ntok,), jnp.int32),
        mesh=mesh,
        compiler_params=pltpu.CompilerParams(needs_layout_passes=False),
        scratch_types=[
            pltpu.VMEM((nchunk,), jnp.float32),
            pltpu.VMEM((nunit,), jnp.int32),
            pltpu.VMEM((nunit, _UNIT), jnp.float32),
            pltpu.VMEM((lat,), jnp.float32),
            pltpu.VMEM((lat,), jnp.float32),
            pltpu.VMEM((rpw + 16,), jnp.float32),
            pltpu.VMEM((256,), jnp.int32),
            pltpu.VMEM((rpw,), jnp.int32),
            pltpu.SemaphoreType.DMA,
        ],
    )(a5, cmax, lvals)
    tkeys2 = tkeys.reshape(ntok, 1)

    bk = 2048 if lat % 2048 == 0 else lat
    nk = lat // bk
    f, xhat = pl.pallas_call(
        functools.partial(_dec_body, nk=nk),
        grid=(ntok // bm, nk),
        in_specs=[
            pl.BlockSpec((bm, bk // _UNIT, _UNIT), lambda i, j: (i, j, 0)),
            pl.BlockSpec((bm, 1), lambda i, j: (i, 0)),
            pl.BlockSpec((vec, bk), lambda i, j: (0, j)),
            pl.BlockSpec((1, vec), lambda i, j: (0, 0)),
        ],
        out_specs=[
            pl.BlockSpec((bm, bk), lambda i, j: (i, j)),
            pl.BlockSpec((bm, vec), lambda i, j: (i, 0)),
        ],
        out_shape=[
            jax.ShapeDtypeStruct((ntok, lat), jnp.float32),
            jax.ShapeDtypeStruct((ntok, vec), jnp.float32),
        ],
        scratch_shapes=[pltpu.VMEM((bm, vec), jnp.float32)],
    )(a, tkeys2, dec_wh, dec_b2)

    return (f, xhat)
